# baseline (speedup 1.0000x reference)
import functools
import math

import jax
import jax.numpy as jnp
from jax import lax
from jax.experimental import pallas as pl
from jax.experimental.pallas import tpu as pltpu
from jax.experimental.pallas import tpu_sc as plsc

NC = 2
NS = 16
NW = NC * NS
LANES = 16
EMB = 64
SCALE = math.sqrt(EMB)
CHUNK = 128


@functools.partial(jax.jit, static_argnames=("total",))
def _lookup(tokens_2d, table, total):
    b_per_w = total // NW
    n_chunks = b_per_w // CHUNK
    vecs_per_chunk = CHUNK * EMB // LANES

    mesh = plsc.VectorSubcoreMesh(core_axis_name="c", subcore_axis_name="s")

    @functools.partial(
        pl.kernel,
        out_type=jax.ShapeDtypeStruct((total, EMB), jnp.float32),
        mesh=mesh,
        scratch_types=[
            pltpu.VMEM((n_chunks, CHUNK), jnp.int32),
            pltpu.VMEM((CHUNK, EMB), jnp.float32),
            pltpu.VMEM((CHUNK, EMB), jnp.float32),
            pltpu.SemaphoreType.DMA,
            pltpu.SemaphoreType.DMA,
        ],
        compiler_params=pltpu.CompilerParams(use_tc_tiling_on_sc=False),
    )
    def body(tok_hbm, table_hbm, out_hbm, idx_v, rows0, rows1, sem0, sem1):
        wid = lax.axis_index("s") * NC + lax.axis_index("c")
        chunk0 = wid * n_chunks
        base = wid * b_per_w

        pltpu.sync_copy(tok_hbm.at[pl.ds(chunk0, n_chunks)], idx_v)

        rows = (rows0, rows1)
        sems = (sem0, sem1)

        def start_gather(i, b):
            pltpu.async_copy(table_hbm.at[idx_v.at[i]], rows[b], sems[b])

        def finish_chunk(i, b):
            pltpu.make_async_copy(
                table_hbm.at[idx_v.at[i]], rows[b], sems[b]
            ).wait()

            @pl.loop(0, vecs_per_chunk)
            def _scale(j):
                r = j // (EMB // LANES)
                c = (j % (EMB // LANES)) * LANES
                rows[b][r, pl.ds(c, LANES)] = rows[b][r, pl.ds(c, LANES)] * SCALE

            pltpu.sync_copy(rows[b], out_hbm.at[pl.ds(base + i * CHUNK, CHUNK)])

        start_gather(0, 0)

        @pl.loop(0, n_chunks, step=2)
        def _chunks(i):
            start_gather(i + 1, 1)
            finish_chunk(i, 0)

            @pl.when(i + 2 < n_chunks)
            def _():
                start_gather(i + 2, 0)

            finish_chunk(i + 1, 1)

    return body(tokens_2d, table)


def kernel(tokens, table):
    shape = tokens.shape
    total = tokens.size
    tok = tokens.reshape(total // CHUNK, CHUNK).astype(jnp.int32)
    out = _lookup(tok, table, total)
    return out.reshape(*shape, EMB)


# speedup vs baseline: 1.4124x; 1.4124x over previous
import functools
import math

import jax
import jax.numpy as jnp
from jax import lax
from jax.experimental import pallas as pl
from jax.experimental.pallas import tpu as pltpu
from jax.experimental.pallas import tpu_sc as plsc

NC = 2
NS = 16
NW = NC * NS
LANES = 16
EMB = 64
SCALE = math.sqrt(EMB)
CHUNK = 128
NBUF = 4


@functools.partial(jax.jit, static_argnames=("total",))
def _lookup(tokens_2d, table, total):
    b_per_w = total // NW
    n_chunks = b_per_w // CHUNK

    mesh = plsc.VectorSubcoreMesh(core_axis_name="c", subcore_axis_name="s")

    row_bufs = [pltpu.VMEM((CHUNK, EMB), jnp.float32) for _ in range(NBUF)]
    gsems = [pltpu.SemaphoreType.DMA for _ in range(NBUF)]
    ssems = [pltpu.SemaphoreType.DMA for _ in range(NBUF)]

    @functools.partial(
        pl.kernel,
        out_type=jax.ShapeDtypeStruct((total, EMB), jnp.float32),
        mesh=mesh,
        scratch_types=[pltpu.VMEM((n_chunks, CHUNK), jnp.int32)]
        + row_bufs + gsems + ssems,
        compiler_params=pltpu.CompilerParams(use_tc_tiling_on_sc=False),
    )
    def body(tok_hbm, table_hbm, out_hbm, idx_v, *bufs_and_sems):
        rows = bufs_and_sems[:NBUF]
        gsem = bufs_and_sems[NBUF:2 * NBUF]
        ssem = bufs_and_sems[2 * NBUF:3 * NBUF]

        wid = lax.axis_index("s") * NC + lax.axis_index("c")
        chunk0 = wid * n_chunks
        base = wid * b_per_w

        pltpu.sync_copy(tok_hbm.at[pl.ds(chunk0, n_chunks)], idx_v)

        def start_gather(i, b):
            pltpu.async_copy(table_hbm.at[idx_v.at[i]], rows[b], gsem[b])

        def wait_gather(i, b):
            pltpu.make_async_copy(
                table_hbm.at[idx_v.at[i]], rows[b], gsem[b]
            ).wait()

        def out_slice(i):
            return out_hbm.at[pl.ds(base + i * CHUNK, CHUNK)]

        def start_store(i, b):
            pltpu.async_copy(rows[b], out_slice(i), ssem[b])

        def wait_store(i, b):
            pltpu.make_async_copy(rows[b], out_slice(i), ssem[b]).wait()

        def scale(b):
            @pl.loop(0, CHUNK, unroll=4)
            def _rows(r):
                for c in range(EMB // LANES):
                    sl = pl.ds(c * LANES, LANES)
                    rows[b][r, sl] = rows[b][r, sl] * SCALE

        for b in range(NBUF):
            start_gather(b, b)

        @pl.loop(0, n_chunks, step=NBUF)
        def _ring(g):
            for b in range(NBUF):
                c = g + b
                wait_gather(c, b)
                scale(b)
                start_store(c, b)

                @pl.when(c + NBUF < n_chunks)
                def _():
                    wait_store(c, b)
                    start_gather(c + NBUF, b)

        for b in range(NBUF):
            c = n_chunks - NBUF + b
            wait_store(c, b)

    return body(tokens_2d, table)


def kernel(tokens, table):
    shape = tokens.shape
    total = tokens.size
    tok = tokens.reshape(total // CHUNK, CHUNK).astype(jnp.int32)
    out = _lookup(tok, table, total)
    return out.reshape(*shape, EMB)
